# baseline (speedup 1.0000x reference)
import functools

import jax
import jax.numpy as jnp
from jax import lax
from jax.experimental import pallas as pl
from jax.experimental.pallas import tpu as pltpu
from jax.experimental.pallas import tpu_sc as plsc


def kernel(x, row_embed, col_embed):
    b, c, h, w = x.shape
    ch = c // 2
    hw = h * w
    L = 16
    NW = 32
    S = hw // NW
    SE = S * c
    K = 2
    xt = jnp.transpose(x, (0, 2, 3, 1)).reshape(b * hw * c)
    row_flat = row_embed.reshape(-1)
    col_flat = col_embed.reshape(-1)
    tab_e = h * ch

    mesh = plsc.VectorSubcoreMesh(core_axis_name="c", subcore_axis_name="s")

    @functools.partial(
        pl.kernel,
        mesh=mesh,
        out_type=jax.ShapeDtypeStruct((b * hw * c,), jnp.float32),
        scratch_types=[
            pltpu.VMEM((tab_e,), jnp.float32),
            pltpu.VMEM((tab_e,), jnp.float32),
            pltpu.VMEM((SE,), jnp.float32),
            pltpu.VMEM((K, SE), jnp.float32),
            pltpu.VMEM((K, SE), jnp.float32),
            pltpu.SemaphoreType.DMA((K,)),
            pltpu.SemaphoreType.DMA((K,)),
        ],
    )
    def _sc_kernel(x_hbm, row_hbm, col_hbm, out_hbm,
                   row_v, col_v, pos_v, in_bufs, out_bufs, in_sems, out_sems):
        wid = lax.axis_index("s") * 2 + lax.axis_index("c")
        base = wid * SE

        pltpu.sync_copy(row_hbm.at[pl.ds(0, tab_e)], row_v)
        pltpu.sync_copy(col_hbm.at[pl.ds(0, tab_e)], col_v)

        for j in range(S):
            l = wid * S + j
            h_off = (l // w) * ch
            w_off = (l % w) * ch
            for kk in range(ch // L):
                pos_v[pl.ds(j * c + kk * L, L)] = row_v[pl.ds(h_off + kk * L, L)]
                pos_v[pl.ds(j * c + ch + kk * L, L)] = col_v[pl.ds(w_off + kk * L, L)]

        def in_copy(bi, slot):
            return pltpu.make_async_copy(
                x_hbm.at[pl.ds(bi * (hw * c) + base, SE)], in_bufs.at[slot],
                in_sems.at[slot])

        def out_copy(bi, slot):
            return pltpu.make_async_copy(
                out_bufs.at[slot], out_hbm.at[pl.ds(bi * (hw * c) + base, SE)],
                out_sems.at[slot])

        for k in range(K):
            in_copy(k, k).start()

        def body(bi, _):
            slot = lax.rem(bi, K)
            in_copy(bi, slot).wait()

            @pl.when(bi >= K)
            def _():
                out_copy(bi - K, slot).wait()

            for q in range(SE // L):
                sl = pl.ds(q * L, L)
                out_bufs[slot, sl] = in_bufs[slot, sl] + pos_v[sl]

            out_copy(bi, slot).start()

            @pl.when(bi + K < b)
            def _():
                in_copy(bi + K, slot).start()

            return 0

        lax.fori_loop(0, b, body, 0)

        def drain(bi, _):
            out_copy(bi, lax.rem(bi, K)).wait()
            return 0

        lax.fori_loop(b - K, b, drain, 0)

    out = _sc_kernel(xt, row_flat, col_flat)
    return out.reshape(b, h, w, c).transpose(0, 3, 1, 2)


# speedup vs baseline: 9.7426x; 9.7426x over previous
import functools

import jax
import jax.numpy as jnp
from jax.experimental import pallas as pl
from jax.experimental.pallas import tpu as pltpu


def _posenc_kernel(x_ref, row_ref, col_ref, o_ref, pos_ref, *, H, W, CH):
    i = pl.program_id(0)
    HW = H * W

    @pl.when(i == 0)
    def _build_pos():
        r = jax.lax.broadcasted_iota(jnp.int32, (HW, H), 0)
        k = jax.lax.broadcasted_iota(jnp.int32, (HW, H), 1)
        Eh = (r // W == k).astype(jnp.float32)
        Ew = (r % W == k).astype(jnp.float32)
        pos_ref[:, :CH] = jax.lax.dot(
            Eh, row_ref[:H, :], precision=jax.lax.Precision.HIGHEST,
            preferred_element_type=jnp.float32)
        pos_ref[:, CH:] = jax.lax.dot(
            Ew, col_ref[:W, :], precision=jax.lax.Precision.HIGHEST,
            preferred_element_type=jnp.float32)

    o_ref[...] = x_ref[...] + pos_ref[...][None]


def kernel(x, row_embed, col_embed):
    b, c, h, w = x.shape
    ch = c // 2
    hw = h * w
    G = 16
    xt = jnp.transpose(x, (0, 2, 3, 1)).reshape(b, hw, c)
    body = functools.partial(_posenc_kernel, H=h, W=w, CH=ch)
    out = pl.pallas_call(
        body,
        grid=(b // G,),
        in_specs=[
            pl.BlockSpec((G, hw, c), lambda i: (i, 0, 0)),
            pl.BlockSpec(row_embed.shape, lambda i: (0, 0)),
            pl.BlockSpec(col_embed.shape, lambda i: (0, 0)),
        ],
        out_specs=pl.BlockSpec((G, hw, c), lambda i: (i, 0, 0)),
        out_shape=jax.ShapeDtypeStruct((b, hw, c), x.dtype),
        scratch_shapes=[pltpu.VMEM((hw, c), jnp.float32)],
    )(xt, row_embed, col_embed)
    return out.reshape(b, h, w, c).transpose(0, 3, 1, 2)
